# Initial kernel scaffold; baseline (speedup 1.0000x reference)
import jax
import jax.numpy as jnp
from jax.experimental import pallas as pl


def kernel(inputs, W):
    raise NotImplementedError("write your pallas kernel here")


# speedup vs baseline: 1.3983x; 1.3983x over previous
import functools

import jax
import jax.numpy as jnp
from jax import lax
from jax.experimental import pallas as pl
from jax.experimental.pallas import tpu as pltpu
from jax.experimental.pallas import tpu_sc as plsc

NE = 8192
D = 256
B = 8
L = 1024
TOK = B * L
CCOST = 0.25
WIN = 4096

TM = 512
LPB = L // TM
NI = TOK // TM


def _argmin_body(x_ref, w_ref, xsq_ref, wsq_ref, idx_ref, loss_ref, acc):
    i = pl.program_id(0)
    x_blk = x_ref[0]
    w_blk = w_ref[...]
    m = lax.dot_general(w_blk, x_blk, (((1,), (0,)), ((), ())),
                        preferred_element_type=jnp.float32)
    dist = (xsq_ref[...][None, :] + wsq_ref[...][:, None]) - 2.0 * m

    def rnd(v):
        return v.astype(jnp.bfloat16).astype(jnp.float32)

    def wmin(lo, hi):
        sub = dist[lo:hi]
        return (jnp.min(sub, axis=0),
                jnp.argmin(sub, axis=0).astype(jnp.int32) + lo)

    bounds = [0, WIN, NE]
    parts = [wmin(lo, hi) for lo, hi in zip(bounds[:-1], bounds[1:])]
    v, a = rnd(parts[0][0]), parts[0][1]
    for k in range(1, len(parts)):
        qv, qa = parts[k]
        better = (qv < v) | ((qv == v) & (qa < a))
        v = rnd(jnp.where(better, qv, v))
        a = jnp.where(better, qa, a)
    idx_ref[...] = a

    part = jnp.sum(jnp.min(dist, axis=0))

    @pl.when(i == 0)
    def _():
        acc[0] = part

    @pl.when(i > 0)
    def _():
        acc[0] = acc[0] + part

    @pl.when(i == NI - 1)
    def _():
        loss_ref[0, 0] = CCOST * acc[0] / jnp.float32(TOK * D)


def _argmin_stage(inputs, W, xsq, wsq):
    return pl.pallas_call(
        _argmin_body,
        grid=(NI,),
        in_specs=[
            pl.BlockSpec((1, D, TM), lambda i: (i // LPB, 0, i % LPB)),
            pl.BlockSpec((NE, D), lambda i: (0, 0)),
            pl.BlockSpec((TM,), lambda i: (i,)),
            pl.BlockSpec((NE,), lambda i: (0,)),
        ],
        out_specs=[
            pl.BlockSpec((TM,), lambda i: (i,)),
            pl.BlockSpec((1, 1), lambda i: (0, 0), memory_space=pltpu.SMEM),
        ],
        out_shape=[
            jax.ShapeDtypeStruct((TOK,), jnp.int32),
            jax.ShapeDtypeStruct((1, 1), jnp.float32),
        ],
        scratch_shapes=[pltpu.SMEM((1,), jnp.float32)],
        compiler_params=pltpu.CompilerParams(
            dimension_semantics=("arbitrary",)),
    )(inputs, W, xsq, wsq)


def _sc_gather(W, idx):
    info = plsc.get_sparse_core_info()
    nw = info.num_cores * info.num_subcores
    bpw = TOK // nw
    mesh = plsc.VectorSubcoreMesh(core_axis_name="c", subcore_axis_name="s")

    @functools.partial(
        pl.kernel, mesh=mesh,
        out_type=jax.ShapeDtypeStruct((TOK, D), jnp.float32),
        scratch_types=[
            pltpu.VMEM((bpw,), jnp.int32),
            pltpu.VMEM((bpw, D), jnp.float32),
            pltpu.SemaphoreType.DMA,
        ],
    )
    def k(table_hbm, idx_hbm, out_hbm, idx_v, rows_v, sem):
        wid = lax.axis_index("s") * info.num_cores + lax.axis_index("c")
        base = wid * bpw
        pltpu.sync_copy(idx_hbm.at[pl.ds(base, bpw)], idx_v)
        pltpu.async_copy(table_hbm.at[idx_v], rows_v, sem).wait()
        pltpu.sync_copy(rows_v, out_hbm.at[pl.ds(base, bpw)])

    return k(W, idx)


TM2 = 512
TC2 = 2048
NI2 = TOK // TM2
NJ2 = NE // TC2


def _onehot_body(idx_ref, enc_ref, plex_ref, counts):
    j = pl.program_id(0)
    i = pl.program_id(1)
    idx_blk = idx_ref[...]
    cols = lax.broadcasted_iota(jnp.int32, (TM2, TC2), 1) + j * TC2
    enc = (cols == idx_blk[:, None]).astype(jnp.float32)
    enc_ref[...] = enc

    @pl.when(i == 0)
    def _():
        counts[pl.ds(j * TC2, TC2)] = jnp.sum(enc, axis=0)

    @pl.when(i > 0)
    def _():
        counts[pl.ds(j * TC2, TC2)] = (
            counts[pl.ds(j * TC2, TC2)] + jnp.sum(enc, axis=0))

    @pl.when((i == NI2 - 1) & (j == NJ2 - 1))
    def _():
        p = counts[...] * jnp.float32(1.0 / TOK)
        plex_ref[0, 0] = jnp.exp(-jnp.sum(p * jnp.log(p + 1e-10)))


def _onehot_stage(idx):
    return pl.pallas_call(
        _onehot_body,
        grid=(NJ2, NI2),
        in_specs=[pl.BlockSpec((TM2,), lambda j, i: (i,))],
        out_specs=[
            pl.BlockSpec((TM2, TC2), lambda j, i: (i, j)),
            pl.BlockSpec((1, 1), lambda j, i: (0, 0),
                         memory_space=pltpu.SMEM),
        ],
        out_shape=[
            jax.ShapeDtypeStruct((TOK, NE), jnp.float32),
            jax.ShapeDtypeStruct((1, 1), jnp.float32),
        ],
        scratch_shapes=[pltpu.VMEM((NE,), jnp.float32)],
        compiler_params=pltpu.CompilerParams(
            dimension_semantics=("arbitrary", "arbitrary")),
    )(idx)


def kernel(inputs, W):
    x = jnp.transpose(inputs, (0, 2, 1)).reshape(-1, D)
    xsq = (x**2).sum(1)
    wsq = (W**2).sum(1)
    idx, loss = _argmin_stage(inputs, W, xsq, wsq)
    quantized = _sc_gather(W, idx)
    encodings, plex = _onehot_stage(idx)
    quantized_out = jnp.transpose(quantized.reshape(B, L, D), (0, 2, 1))
    return (loss.reshape(()), quantized_out, plex.reshape(()),
            W, idx[:, None], encodings)
